# baseline (speedup 1.0000x reference)
import functools

import jax
import jax.numpy as jnp
from jax import lax
from jax.experimental import pallas as pl
from jax.experimental.pallas import tpu as pltpu
from jax.experimental.pallas import tpu_sc as plsc

NUM_SC = 2
NUM_SUBCORES = 16
NW = NUM_SC * NUM_SUBCORES


def _node_mask_sc(edge_index, n_pad):
    e = edge_index.shape[1]
    step = (e // NW) // 128 * 128
    wlen = e - (NW - 1) * step
    per_tile = n_pad // NUM_SUBCORES
    mesh = plsc.VectorSubcoreMesh(core_axis_name="c", subcore_axis_name="s")

    @functools.partial(
        pl.kernel,
        out_type=jax.ShapeDtypeStruct((NUM_SC, n_pad), jnp.int32),
        mesh=mesh,
        scratch_types=[
            pltpu.VMEM((2, wlen), jnp.int32),
            pltpu.VMEM((wlen,), jnp.int32),
            pltpu.VMEM((per_tile,), jnp.int32),
            pltpu.VMEM_SHARED((n_pad,), jnp.int32),
            pltpu.SemaphoreType.DMA,
        ],
    )
    def mask_kernel(edges_hbm, out_hbm, win_v, idx_v, z_v, acc_sh, sem):
        c = lax.axis_index("c")
        s = lax.axis_index("s")
        wid = s * NUM_SC + c
        my_base = pl.multiple_of(s * per_tile, 8)
        win_base = pl.multiple_of(wid * step, 128)

        h = pltpu.async_copy(edges_hbm.at[:, pl.ds(win_base, wlen)], win_v, sem)
        for i in range(per_tile // 16):
            z_v[pl.ds(i * 16, 16)] = jnp.full((16,), -1, jnp.int32)
        pltpu.sync_copy(z_v, acc_sh.at[pl.ds(my_base, per_tile)])
        h.wait()

        def copy_dst_row(j, carry):
            for u in range(4):
                o = j * 64 + u * 16
                idx_v[pl.ds(o, 16)] = win_v[1, pl.ds(o, 16)]
            return carry

        lax.fori_loop(0, wlen // 64, copy_dst_row, 0)
        plsc.subcore_barrier()
        pltpu.sync_copy(idx_v, acc_sh.at[idx_v])
        plsc.subcore_barrier()
        pltpu.sync_copy(
            acc_sh.at[pl.ds(my_base, per_tile)],
            out_hbm.at[c, pl.ds(my_base, per_tile)],
        )

    return mask_kernel(edge_index)


def _masked_matmul_tc(x, w, mask_t, blk):
    n, in_f = x.shape
    hf = w.shape[0]

    def mm_kernel(x_ref, w_ref, m_ref, o_ref):
        y = lax.dot_general(
            x_ref[...], w_ref[...], (((1,), (1,)), ((), ())),
            preferred_element_type=jnp.float32,
        )
        d = jnp.maximum(m_ref[:, 0:1], m_ref[:, 1:2])
        o_ref[...] = jnp.where(d >= 0, y, 0.0)

    return pl.pallas_call(
        mm_kernel,
        grid=(n // blk,),
        in_specs=[
            pl.BlockSpec((blk, in_f), lambda i: (i, 0)),
            pl.BlockSpec((hf, in_f), lambda i: (0, 0)),
            pl.BlockSpec((blk, NUM_SC), lambda i: (i, 0)),
        ],
        out_specs=pl.BlockSpec((blk, hf), lambda i: (i, 0)),
        out_shape=jax.ShapeDtypeStruct((n, hf), jnp.float32),
    )(x, w, mask_t)


def kernel(x, edge_index, W, a):
    del a
    n = x.shape[0]
    blk = 10000
    n_pad = ((n + 255) // 256) * 256
    mask = _node_mask_sc(edge_index, n_pad)
    return _masked_matmul_tc(x, W, mask.T, blk)


# speedup vs baseline: 1.0540x; 1.0540x over previous
import functools

import jax
import jax.numpy as jnp
from jax import lax
from jax.experimental import pallas as pl
from jax.experimental.pallas import tpu as pltpu
from jax.experimental.pallas import tpu_sc as plsc

NUM_SC = 2
NUM_SUBCORES = 16
NW = NUM_SC * NUM_SUBCORES


def _node_mask_sc(edge_index, n_pad):
    e = edge_index.shape[1]
    step = (e // NW) // 128 * 128
    wlen = e - (NW - 1) * step
    per_tile = n_pad // NUM_SUBCORES
    mesh = plsc.VectorSubcoreMesh(core_axis_name="c", subcore_axis_name="s")

    @functools.partial(
        pl.kernel,
        out_type=jax.ShapeDtypeStruct((NUM_SC, n_pad), jnp.int32),
        mesh=mesh,
        scratch_types=[
            pltpu.VMEM((2, wlen), jnp.int32),
            pltpu.VMEM((wlen // 2,), jnp.int32),
            pltpu.VMEM((wlen // 2,), jnp.int32),
            pltpu.VMEM((per_tile,), jnp.int32),
            pltpu.VMEM_SHARED((n_pad,), jnp.int32),
            pltpu.SemaphoreType.DMA,
        ],
    )
    def mask_kernel(edges_hbm, out_hbm, win_v, idx_a, idx_b, z_v, acc_sh, sem):
        c = lax.axis_index("c")
        s = lax.axis_index("s")
        wid = s * NUM_SC + c
        my_base = pl.multiple_of(s * per_tile, 8)
        win_base = pl.multiple_of(wid * step, 128)

        h = pltpu.async_copy(edges_hbm.at[:, pl.ds(win_base, wlen)], win_v, sem)
        for i in range(per_tile // 16):
            z_v[pl.ds(i * 16, 16)] = jnp.full((16,), -1, jnp.int32)
        pltpu.sync_copy(z_v, acc_sh.at[pl.ds(my_base, per_tile)])
        h.wait()

        half = wlen // 2

        def copy_dst_row(j, carry):
            for u in range(4):
                o = j * 64 + u * 16
                idx_a[pl.ds(o, 16)] = win_v[1, pl.ds(o, 16)]
                idx_b[pl.ds(o, 16)] = win_v[1, pl.ds(half + o, 16)]
            return carry

        lax.fori_loop(0, half // 64, copy_dst_row, 0)
        plsc.subcore_barrier()
        h1 = pltpu.async_copy(idx_a, acc_sh.at[idx_a], sem)
        h2 = pltpu.async_copy(idx_b, acc_sh.at[idx_b], sem)
        h1.wait()
        h2.wait()
        plsc.subcore_barrier()
        pltpu.sync_copy(
            acc_sh.at[pl.ds(my_base, per_tile)],
            out_hbm.at[c, pl.ds(my_base, per_tile)],
        )

    return mask_kernel(edge_index)


def _masked_matmul_tc(x, w, mask_t, blk):
    n, in_f = x.shape
    hf = w.shape[0]

    def mm_kernel(x_ref, w_ref, m_ref, o_ref):
        y = lax.dot_general(
            x_ref[...], w_ref[...], (((1,), (1,)), ((), ())),
            preferred_element_type=jnp.float32,
        )
        d = jnp.maximum(m_ref[:, 0:1], m_ref[:, 1:2])
        o_ref[...] = jnp.where(d >= 0, y, 0.0)

    return pl.pallas_call(
        mm_kernel,
        grid=(n // blk,),
        in_specs=[
            pl.BlockSpec((blk, in_f), lambda i: (i, 0)),
            pl.BlockSpec((hf, in_f), lambda i: (0, 0)),
            pl.BlockSpec((blk, NUM_SC), lambda i: (i, 0)),
        ],
        out_specs=pl.BlockSpec((blk, hf), lambda i: (i, 0)),
        out_shape=jax.ShapeDtypeStruct((n, hf), jnp.float32),
    )(x, w, mask_t)


def kernel(x, edge_index, W, a):
    del a
    n = x.shape[0]
    blk = 5000
    n_pad = ((n + 255) // 256) * 256
    mask = _node_mask_sc(edge_index, n_pad)
    return _masked_matmul_tc(x, W, mask.T, blk)


# speedup vs baseline: 1.1511x; 1.0922x over previous
import functools

import jax
import jax.numpy as jnp
from jax import lax
from jax.experimental import pallas as pl
from jax.experimental.pallas import tpu as pltpu
from jax.experimental.pallas import tpu_sc as plsc

NUM_SC = 2
NUM_SUBCORES = 16
NW = NUM_SC * NUM_SUBCORES


def _node_mask_sc(edge_index, n_pad):
    e = edge_index.shape[1]
    step = (e // NW) // 128 * 128
    wlen = e - (NW - 1) * step
    per_tile = n_pad // NUM_SUBCORES
    mesh = plsc.VectorSubcoreMesh(core_axis_name="c", subcore_axis_name="s")

    @functools.partial(
        pl.kernel,
        out_type=jax.ShapeDtypeStruct((NUM_SC, n_pad), jnp.int32),
        mesh=mesh,
        scratch_types=[
            pltpu.VMEM((2, wlen), jnp.int32),
            pltpu.VMEM((wlen // 2,), jnp.int32),
            pltpu.VMEM((wlen // 2,), jnp.int32),
            pltpu.VMEM((per_tile,), jnp.int32),
            pltpu.VMEM_SHARED((n_pad,), jnp.int32),
            pltpu.SemaphoreType.DMA,
        ],
    )
    def mask_kernel(edges_hbm, out_hbm, win_v, idx_a, idx_b, z_v, acc_sh, sem):
        c = lax.axis_index("c")
        s = lax.axis_index("s")
        wid = s * NUM_SC + c
        my_base = pl.multiple_of(s * per_tile, 8)
        win_base = pl.multiple_of(wid * step, 128)

        h = pltpu.async_copy(edges_hbm.at[:, pl.ds(win_base, wlen)], win_v, sem)
        for i in range(per_tile // 16):
            z_v[pl.ds(i * 16, 16)] = jnp.full((16,), -1, jnp.int32)
        pltpu.sync_copy(z_v, acc_sh.at[pl.ds(my_base, per_tile)])
        h.wait()

        half = wlen // 2

        def copy_dst_row(j, carry):
            for u in range(4):
                o = j * 64 + u * 16
                idx_a[pl.ds(o, 16)] = win_v[1, pl.ds(o, 16)]
                idx_b[pl.ds(o, 16)] = win_v[1, pl.ds(half + o, 16)]
            return carry

        lax.fori_loop(0, half // 64, copy_dst_row, 0)
        plsc.subcore_barrier()
        h1 = pltpu.async_copy(idx_a, acc_sh.at[idx_a], sem)
        h2 = pltpu.async_copy(idx_b, acc_sh.at[idx_b], sem)
        h1.wait()
        h2.wait()
        plsc.subcore_barrier()
        pltpu.sync_copy(
            acc_sh.at[pl.ds(my_base, per_tile)],
            out_hbm.at[c, pl.ds(my_base, per_tile)],
        )

    return mask_kernel(edge_index)


def _masked_matmul_tc(x, w, mask, blk):
    n, in_f = x.shape
    hf = w.shape[0]
    n_pad = mask.shape[1]

    def mm_kernel(x_ref, w_ref, m_ref, o_ref, mt_ref):
        i = pl.program_id(0)

        @pl.when(i == 0)
        def _():
            mt_ref[...] = jnp.transpose(m_ref[...], (1, 0))

        y = lax.dot_general(
            x_ref[...], w_ref[...], (((1,), (1,)), ((), ())),
            preferred_element_type=jnp.float32,
        )
        rows = pl.ds(i * blk, blk)
        d = jnp.maximum(mt_ref[rows, 0:1], mt_ref[rows, 1:2])
        o_ref[...] = jnp.where(d >= 0, y, 0.0)

    return pl.pallas_call(
        mm_kernel,
        grid=(n // blk,),
        in_specs=[
            pl.BlockSpec((blk, in_f), lambda i: (i, 0)),
            pl.BlockSpec((hf, in_f), lambda i: (0, 0)),
            pl.BlockSpec((NUM_SC, n_pad), lambda i: (0, 0)),
        ],
        out_specs=pl.BlockSpec((blk, hf), lambda i: (i, 0)),
        out_shape=jax.ShapeDtypeStruct((n, hf), jnp.float32),
        scratch_shapes=[pltpu.VMEM((n_pad, NUM_SC), jnp.int32)],
    )(x, w, mask)


def kernel(x, edge_index, W, a):
    del a
    n = x.shape[0]
    blk = 5000
    n_pad = ((n + 255) // 256) * 256
    mask = _node_mask_sc(edge_index, n_pad)
    return _masked_matmul_tc(x, W, mask, blk)
